# baseline (speedup 1.0000x reference)
import functools

import jax
import jax.numpy as jnp
from jax import lax
from jax.experimental import pallas as pl
from jax.experimental.pallas import tpu as pltpu
from jax.experimental.pallas import tpu_sc as plsc

SIZE_UU = 1000
SIZE_UP = 100000
D = 128

BATCH = 4096
NNEG = 10

NC = 2
NS = 16
NW = NC * NS
PER_W = BATCH // NW
SUBC = 64
NSUB = PER_W // SUBC
NGRP = SUBC // 16
NDOT = 1 + NNEG

BS = 2000
NB = SIZE_UP // BS
NSLOT = 4


CH = 32
NCH = PER_W // CH
NROW = 2 + NNEG
UNROLL = 4
SCAL = NDOT * CH
ROWB = NROW * CH
UU_PAD = 3 * 128
RP_PAD = 4 * 128
UU_WORDS = NCH * UU_PAD
RP_WORDS = NCH * RP_PAD
IDX_WORDS = UU_WORDS + 2 * RP_WORDS


DW = D // 2


def _sc_dots_body(
    g_uu, up_nodes, up_ctx, pp_nodes, pp_ctx, idx_all,
    out_hbm,
    idx_v, sbuf0, sbuf1, buf0, buf1, dots_v,
    sem0, sem1,
):
  wid = lax.axis_index("s") * NC + lax.axis_index("c")
  lane = lax.broadcasted_iota(jnp.int32, (16,), 0)

  pltpu.sync_copy(idx_all.at[pl.ds(wid, 1)], idx_v)

  chunks = [("uu", c) for c in range(NCH)]
  chunks += [(l, c) for l in (1, 2) for c in range(NCH)]
  sbufs = (sbuf0, sbuf1)
  bufs = (buf0, buf1)
  sems = (sem0, sem1)

  def fire(p):
    kind, c = chunks[p]
    s = p % 2
    cps = []
    if kind == "uu":
      base = c * UU_PAD
      for off, n in ((0, 128), (128, 128), (256, 96)):
        idx_ref = idx_v.at[0, pl.ds(base + off, n)]
        cps.append(pltpu.async_copy(
            g_uu.at[idx_ref], sbufs[s].at[pl.ds(off, n)], sems[s]))
    else:
      nodes, ctx = (up_nodes, up_ctx) if kind == 1 else (pp_nodes, pp_ctx)
      base = UU_WORDS + (kind - 1) * RP_WORDS + c * RP_PAD
      cps.append(pltpu.async_copy(
          nodes.at[idx_v.at[0, pl.ds(base, CH)]],
          bufs[s].at[pl.ds(0, CH)], sems[s]))
      for off, n in ((0, 128), (128, 128), (256, 96)):
        idx_ref = idx_v.at[0, pl.ds(base + 128 + off, n)]
        cps.append(pltpu.async_copy(
            ctx.at[idx_ref], bufs[s].at[pl.ds(CH + off, n)], sems[s]))
    return cps

  pending = fire(0)
  for p in range(len(chunks)):
    s = p % 2
    kind, c = chunks[p]
    nxt = fire(p + 1) if p + 1 < len(chunks) else ()
    for cp in pending:
      cp.wait()
    pending = nxt

    if kind == "uu":
      sbuf = sbufs[s]
      for g in range(CH // 16):
        col = c * CH + g * 16
        dots_v[0, 0, pl.ds(col, 16)] = sbuf[pl.ds(g * 16, 16)]
        for k in range(NNEG):
          dots_v[0, 1 + k, pl.ds(col, 16)] = (
              -sbuf[pl.ds((1 + k) * CH + g * 16, 16)])
      continue

    buf = bufs[s]
    for g in range(CH // 16):
      arow = g * 16 + lane
      crows = [CH + j * CH + g * 16 + lane for j in range(NDOT)]

      def dim_body(t, accs, buf=buf, arow=arow, crows=crows):
        new = accs
        for u in range(UNROLL):
          dd = jnp.full((16,), t * UNROLL + u, dtype=jnp.int32)
          a = plsc.load_gather(buf, [arow, dd])
          new = tuple(
              new[j] + a * plsc.load_gather(buf, [crows[j], dd])
              for j in range(NDOT))
        return new

      accs = lax.fori_loop(
          0, D // UNROLL, dim_body,
          tuple(jnp.zeros((16,), jnp.float32) for _ in range(NDOT)))
      col = c * CH + g * 16
      dots_v[0, 11 * kind + 0, pl.ds(col, 16)] = accs[0]
      for k in range(NNEG):
        dots_v[0, 11 * kind + 1 + k, pl.ds(col, 16)] = -accs[1 + k]

  pltpu.sync_copy(dots_v, out_hbm.at[pl.ds(wid, 1)])


def _sc_dots(tables_and_indices):
  mesh = plsc.VectorSubcoreMesh(core_axis_name="c", subcore_axis_name="s")
  fn = pl.kernel(
      _sc_dots_body,
      out_type=jax.ShapeDtypeStruct((NW, 3 * NDOT, PER_W), jnp.float32),
      mesh=mesh,
      compiler_params=pltpu.CompilerParams(
          needs_layout_passes=False, skip_device_barrier=True),
      scratch_types=[
          pltpu.VMEM((1, IDX_WORDS), jnp.int32),
          pltpu.VMEM((SCAL,), jnp.float32),
          pltpu.VMEM((SCAL,), jnp.float32),
          pltpu.VMEM((ROWB, D), jnp.float32),
          pltpu.VMEM((ROWB, D), jnp.float32),
          pltpu.VMEM((1, 3 * NDOT, PER_W), jnp.float32),
          pltpu.SemaphoreType.DMA,
          pltpu.SemaphoreType.DMA,
      ],
  )
  return fn(*tables_and_indices)


def _gram_body(a_ref, b_ref, o_ref):
  o_ref[...] = lax.dot_general(
      a_ref[...], b_ref[...], (((1,), (1,)), ((), ())),
      preferred_element_type=jnp.float32)


def _uu_gram(uu_nodes, uu_ctx):
  return pl.pallas_call(
      _gram_body,
      out_shape=jax.ShapeDtypeStruct((SIZE_UU, SIZE_UU), jnp.float32),
  )(uu_nodes, uu_ctx)


def _fusion_body(uu_hbm, pp_hbm, up_ref, w_ref, b_ref, v_ref, o_ref,
                 a_sc, sems):
  i = pl.program_id(0)

  def reg_copy(blk, slot):
    return pltpu.make_async_copy(
        pp_hbm.at[pl.ds(blk * BS - 1, BS), :], a_sc.at[slot], sems.at[slot])

  def blk0_copies():
    return (
        pltpu.make_async_copy(
            pp_hbm.at[pl.ds(0, 1), :], a_sc.at[0, pl.ds(0, 1), :],
            sems.at[0]),
        pltpu.make_async_copy(
            uu_hbm.at[pl.ds(1, SIZE_UU - 1), :],
            a_sc.at[0, pl.ds(1, SIZE_UU - 1), :], sems.at[0]),
        pltpu.make_async_copy(
            pp_hbm.at[pl.ds(SIZE_UU - 1, BS - SIZE_UU), :],
            a_sc.at[0, pl.ds(SIZE_UU, BS - SIZE_UU), :], sems.at[0]),
    )

  @pl.when(i == 0)
  def _():
    for cp in blk0_copies():
      cp.start()
    for blk in range(1, min(NSLOT - 1, NB)):
      reg_copy(blk, blk).start()

  @pl.when(i + NSLOT - 1 < NB)
  def _():
    blk = i + NSLOT - 1
    reg_copy(blk, lax.rem(blk, NSLOT)).start()

  slot = lax.rem(i, NSLOT)

  @pl.when(i == 0)
  def _():
    for cp in blk0_copies():
      cp.wait()

  @pl.when(i > 0)
  def _():
    reg_copy(i, slot).wait()

  a = a_sc[slot]
  bmat = up_ref[...]
  w = w_ref[...]
  bias = b_ref[...]
  v = v_ref[...]

  def att_logit(x):
    h = lax.dot_general(x, w, (((1,), (1,)), ((), ())),
                        preferred_element_type=jnp.float32) + bias
    lr = jnp.maximum(h, 0.01 * h)
    return lax.dot_general(lr, v, (((1,), (1,)), ((), ())),
                           preferred_element_type=jnp.float32)

  la = att_logit(a)
  lb = att_logit(bmat)
  m = jnp.maximum(la, lb)
  ea = jnp.exp(la - m)
  eb = jnp.exp(lb - m)
  inv = 1.0 / (ea + eb)
  o_ref[...] = a * (ea * inv) + bmat * (eb * inv)

  @pl.when(i == 0)
  def _():
    o_ref[pl.ds(0, 1), :] = a_sc[0, pl.ds(0, 1), :]


def _fusion(uu_nodes, up_nodes, pp_nodes, att_w, att_b, att_v):
  return pl.pallas_call(
      _fusion_body,
      grid=(NB,),
      in_specs=[
          pl.BlockSpec(memory_space=pl.ANY),
          pl.BlockSpec(memory_space=pl.ANY),
          pl.BlockSpec((BS, D), lambda i: (i, 0)),
          pl.BlockSpec((D, D), lambda i: (0, 0)),
          pl.BlockSpec((1, D), lambda i: (0, 0)),
          pl.BlockSpec((1, D), lambda i: (0, 0)),
      ],
      out_specs=pl.BlockSpec((BS, D), lambda i: (i, 0)),
      out_shape=jax.ShapeDtypeStruct((SIZE_UP, D), jnp.float32),
      scratch_shapes=[
          pltpu.VMEM((NSLOT, BS, D), jnp.float32),
          pltpu.SemaphoreType.DMA((NSLOT,)),
      ],
  )(uu_nodes, pp_nodes, up_nodes, att_w, att_b, att_v)


def _loss_body(dots_ref, o_ref):
  x = dots_ref[...]
  ls = jnp.minimum(x, 0.0) - jnp.log1p(jnp.exp(-jnp.abs(x)))
  o_ref[0, 0] = -jnp.sum(ls) / BATCH


def _loss_epilogue(dots):
  return pl.pallas_call(
      _loss_body,
      out_specs=pl.BlockSpec(memory_space=pltpu.SMEM),
      out_shape=jax.ShapeDtypeStruct((1, 1), jnp.float32),
  )(dots.reshape(3 * NDOT * BATCH // D, D))


def kernel(v_i_uu, v_j_uu, negsamples_uu, v_i_up, v_j_up, negsamples_up,
           v_i_pp, v_j_pp, negsamples_pp, users_history, device,
           emb_uu_nodes, emb_uu_ctx, emb_up_nodes, emb_up_ctx,
           emb_pp_nodes, emb_pp_ctx, att_fc_W, att_fc_b, att_cv_w):
  i32 = jnp.int32

  def chunk_major(rows, pad_to):
    r = rows.shape[0]
    cm = rows.reshape(r, NW, NCH, CH).transpose(1, 2, 0, 3).reshape(
        NW, NCH, r * CH)
    return jnp.pad(cm, ((0, 0), (0, 0), (0, pad_to - r * CH)))

  def pack_rows(vi, vj, negs):
    return jnp.concatenate(
        [vi.astype(i32)[None], vj.astype(i32)[None], negs.astype(i32).T],
        axis=0)

  def pack_rp(vi, vj, negs):
    rows = pack_rows(vi, vj, negs)
    vi_cm = chunk_major(rows[:1], 128)
    ctx_cm = chunk_major(rows[1:], 3 * 128)
    return jnp.concatenate([vi_cm, ctx_cm], axis=2).reshape(NW, RP_WORDS)

  uu_rows = pack_rows(v_i_uu, v_j_uu, negsamples_uu)
  uu_scalar = uu_rows[0][None] * SIZE_UU + uu_rows[1:]

  idx_all = jnp.concatenate([
      chunk_major(uu_scalar, UU_PAD).reshape(NW, UU_WORDS),
      pack_rp(v_i_up, v_j_up, negsamples_up),
      pack_rp(v_i_pp, v_j_pp, negsamples_pp),
  ], axis=1)

  g_uu = _uu_gram(emb_uu_nodes, emb_uu_ctx)
  dots = _sc_dots((
      g_uu.reshape(-1), emb_up_nodes, emb_up_ctx,
      emb_pp_nodes, emb_pp_ctx, idx_all,
  ))
  fused = _fusion(emb_uu_nodes, emb_up_nodes, emb_pp_nodes,
                  att_fc_W, att_fc_b.reshape(1, D), att_cv_w)
  loss = _loss_epilogue(dots)[0, 0]
  return (loss, fused)


# speedup vs baseline: 1.0008x; 1.0008x over previous
import jax
import jax.numpy as jnp
from jax import lax
from jax.experimental import pallas as pl
from jax.experimental.pallas import tpu as pltpu
from jax.experimental.pallas import tpu_sc as plsc

SIZE_UU = 1000
SIZE_UP = 100000
D = 128

BATCH = 4096
NNEG = 10

NC = 2
NS = 16
NW = NC * NS
PER_W = BATCH // NW
NDOT = 1 + NNEG

BS = 2000
NB = SIZE_UP // BS
NSLOT = 4

CH = 32
NCH = PER_W // CH
NROW = 2 + NNEG
UNROLL = 4
SCAL = NDOT * CH
ROWB = NROW * CH
UU_PAD = 3 * 128
RP_PAD = 4 * 128
UU_WORDS = NCH * UU_PAD
RP_WORDS = NCH * RP_PAD
IDX_WORDS = UU_WORDS + 2 * RP_WORDS


def _sc_dots_body(
    g_uu, up_nodes, up_ctx, pp_nodes, pp_ctx, idx_all,
    out_hbm,
    idx_v, sbuf0, sbuf1, buf0, buf1, dots_v,
    sem0, sem1,
):
  wid = lax.axis_index("s") * NC + lax.axis_index("c")
  lane = lax.broadcasted_iota(jnp.int32, (16,), 0)

  pltpu.sync_copy(idx_all.at[pl.ds(wid, 1)], idx_v)

  chunks = [("uu", c) for c in range(NCH)]
  chunks += [(l, c) for l in (1, 2) for c in range(NCH)]
  sbufs = (sbuf0, sbuf1)
  bufs = (buf0, buf1)
  sems = (sem0, sem1)

  def fire(p):
    kind, c = chunks[p]
    s = p % 2
    cps = []
    if kind == "uu":
      base = c * UU_PAD
      for off, n in ((0, 128), (128, 128), (256, 96)):
        idx_ref = idx_v.at[0, pl.ds(base + off, n)]
        cps.append(pltpu.async_copy(
            g_uu.at[idx_ref], sbufs[s].at[pl.ds(off, n)], sems[s]))
    else:
      nodes, ctx = (up_nodes, up_ctx) if kind == 1 else (pp_nodes, pp_ctx)
      base = UU_WORDS + (kind - 1) * RP_WORDS + c * RP_PAD
      cps.append(pltpu.async_copy(
          nodes.at[idx_v.at[0, pl.ds(base, CH)]],
          bufs[s].at[pl.ds(0, CH)], sems[s]))
      for off, n in ((0, 128), (128, 128), (256, 96)):
        idx_ref = idx_v.at[0, pl.ds(base + 128 + off, n)]
        cps.append(pltpu.async_copy(
            ctx.at[idx_ref], bufs[s].at[pl.ds(CH + off, n)], sems[s]))
    return cps

  pending = fire(0)
  for p in range(len(chunks)):
    s = p % 2
    kind, c = chunks[p]
    nxt = fire(p + 1) if p + 1 < len(chunks) else ()
    for cp in pending:
      cp.wait()
    pending = nxt

    if kind == "uu":
      sbuf = sbufs[s]
      for g in range(CH // 16):
        col = c * CH + g * 16
        dots_v[0, 0, pl.ds(col, 16)] = sbuf[pl.ds(g * 16, 16)]
        for k in range(NNEG):
          dots_v[0, 1 + k, pl.ds(col, 16)] = (
              -sbuf[pl.ds((1 + k) * CH + g * 16, 16)])
      continue

    buf = bufs[s]
    for g in range(CH // 16):
      arow = g * 16 + lane
      crows = [CH + j * CH + g * 16 + lane for j in range(NDOT)]

      def dim_body(t, accs, buf=buf, arow=arow, crows=crows):
        new = accs
        for u in range(UNROLL):
          dd = jnp.full((16,), t * UNROLL + u, dtype=jnp.int32)
          a = plsc.load_gather(buf, [arow, dd])
          new = tuple(
              new[j] + a * plsc.load_gather(buf, [crows[j], dd])
              for j in range(NDOT))
        return new

      accs = lax.fori_loop(
          0, D // UNROLL, dim_body,
          tuple(jnp.zeros((16,), jnp.float32) for _ in range(NDOT)))
      col = c * CH + g * 16
      dots_v[0, 11 * kind + 0, pl.ds(col, 16)] = accs[0]
      for k in range(NNEG):
        dots_v[0, 11 * kind + 1 + k, pl.ds(col, 16)] = -accs[1 + k]

  pltpu.sync_copy(dots_v, out_hbm.at[pl.ds(wid, 1)])


def _sc_dots(tables_and_indices):
  mesh = plsc.VectorSubcoreMesh(core_axis_name="c", subcore_axis_name="s")
  fn = pl.kernel(
      _sc_dots_body,
      out_type=jax.ShapeDtypeStruct((NW, 3 * NDOT, PER_W), jnp.float32),
      mesh=mesh,
      compiler_params=pltpu.CompilerParams(needs_layout_passes=False),
      scratch_types=[
          pltpu.VMEM((1, IDX_WORDS), jnp.int32),
          pltpu.VMEM((SCAL,), jnp.float32),
          pltpu.VMEM((SCAL,), jnp.float32),
          pltpu.VMEM((ROWB, D), jnp.float32),
          pltpu.VMEM((ROWB, D), jnp.float32),
          pltpu.VMEM((1, 3 * NDOT, PER_W), jnp.float32),
          pltpu.SemaphoreType.DMA,
          pltpu.SemaphoreType.DMA,
      ],
  )
  return fn(*tables_and_indices)


def _gram_body(a_ref, b_ref, o_ref):
  o_ref[...] = lax.dot_general(
      a_ref[...], b_ref[...], (((1,), (1,)), ((), ())),
      preferred_element_type=jnp.float32)


def _uu_gram(uu_nodes, uu_ctx):
  return pl.pallas_call(
      _gram_body,
      out_shape=jax.ShapeDtypeStruct((SIZE_UU, SIZE_UU), jnp.float32),
  )(uu_nodes, uu_ctx)


def _fusion_body(uu_hbm, pp_hbm, up_ref, w_ref, b_ref, v_ref, o_ref,
                 a_sc, sems):
  i = pl.program_id(0)

  def reg_copy(blk, slot):
    return pltpu.make_async_copy(
        pp_hbm.at[pl.ds(blk * BS - 1, BS), :], a_sc.at[slot], sems.at[slot])

  def blk0_copies():
    return (
        pltpu.make_async_copy(
            pp_hbm.at[pl.ds(0, 1), :], a_sc.at[0, pl.ds(0, 1), :],
            sems.at[0]),
        pltpu.make_async_copy(
            uu_hbm.at[pl.ds(1, SIZE_UU - 1), :],
            a_sc.at[0, pl.ds(1, SIZE_UU - 1), :], sems.at[0]),
        pltpu.make_async_copy(
            pp_hbm.at[pl.ds(SIZE_UU - 1, BS - SIZE_UU), :],
            a_sc.at[0, pl.ds(SIZE_UU, BS - SIZE_UU), :], sems.at[0]),
    )

  @pl.when(i == 0)
  def _():
    for cp in blk0_copies():
      cp.start()
    for blk in range(1, min(NSLOT - 1, NB)):
      reg_copy(blk, blk).start()

  @pl.when(i + NSLOT - 1 < NB)
  def _():
    blk = i + NSLOT - 1
    reg_copy(blk, lax.rem(blk, NSLOT)).start()

  slot = lax.rem(i, NSLOT)

  @pl.when(i == 0)
  def _():
    for cp in blk0_copies():
      cp.wait()

  @pl.when(i > 0)
  def _():
    reg_copy(i, slot).wait()

  a = a_sc[slot]
  bmat = up_ref[...]
  w = w_ref[...]
  bias = b_ref[...]
  v = v_ref[...]

  def att_logit(x):
    h = lax.dot_general(x, w, (((1,), (1,)), ((), ())),
                        preferred_element_type=jnp.float32) + bias
    lr = jnp.maximum(h, 0.01 * h)
    return lax.dot_general(lr, v, (((1,), (1,)), ((), ())),
                           preferred_element_type=jnp.float32)

  la = att_logit(a)
  lb = att_logit(bmat)
  m = jnp.maximum(la, lb)
  ea = jnp.exp(la - m)
  eb = jnp.exp(lb - m)
  inv = 1.0 / (ea + eb)
  o_ref[...] = a * (ea * inv) + bmat * (eb * inv)

  @pl.when(i == 0)
  def _():
    o_ref[pl.ds(0, 1), :] = a_sc[0, pl.ds(0, 1), :]


def _fusion(uu_nodes, up_nodes, pp_nodes, att_w, att_b, att_v):
  return pl.pallas_call(
      _fusion_body,
      grid=(NB,),
      in_specs=[
          pl.BlockSpec(memory_space=pl.ANY),
          pl.BlockSpec(memory_space=pl.ANY),
          pl.BlockSpec((BS, D), lambda i: (i, 0)),
          pl.BlockSpec((D, D), lambda i: (0, 0)),
          pl.BlockSpec((1, D), lambda i: (0, 0)),
          pl.BlockSpec((1, D), lambda i: (0, 0)),
      ],
      out_specs=pl.BlockSpec((BS, D), lambda i: (i, 0)),
      out_shape=jax.ShapeDtypeStruct((SIZE_UP, D), jnp.float32),
      scratch_shapes=[
          pltpu.VMEM((NSLOT, BS, D), jnp.float32),
          pltpu.SemaphoreType.DMA((NSLOT,)),
      ],
  )(uu_nodes, pp_nodes, up_nodes, att_w, att_b, att_v)


def _loss_body(dots_ref, o_ref):
  x = dots_ref[...]
  ls = jnp.minimum(x, 0.0) - jnp.log1p(jnp.exp(-jnp.abs(x)))
  o_ref[0, 0] = -jnp.sum(ls) / BATCH


def _loss_epilogue(dots):
  return pl.pallas_call(
      _loss_body,
      out_specs=pl.BlockSpec(memory_space=pltpu.SMEM),
      out_shape=jax.ShapeDtypeStruct((1, 1), jnp.float32),
  )(dots.reshape(3 * NDOT * BATCH // D, D))


def kernel(v_i_uu, v_j_uu, negsamples_uu, v_i_up, v_j_up, negsamples_up,
           v_i_pp, v_j_pp, negsamples_pp, users_history, device,
           emb_uu_nodes, emb_uu_ctx, emb_up_nodes, emb_up_ctx,
           emb_pp_nodes, emb_pp_ctx, att_fc_W, att_fc_b, att_cv_w):
  i32 = jnp.int32

  def chunk_major(rows, pad_to):
    r = rows.shape[0]
    cm = rows.reshape(r, NW, NCH, CH).transpose(1, 2, 0, 3).reshape(
        NW, NCH, r * CH)
    return jnp.pad(cm, ((0, 0), (0, 0), (0, pad_to - r * CH)))

  def pack_rows(vi, vj, negs):
    return jnp.concatenate(
        [vi.astype(i32)[None], vj.astype(i32)[None], negs.astype(i32).T],
        axis=0)

  def pack_rp(vi, vj, negs):
    rows = pack_rows(vi, vj, negs)
    vi_cm = chunk_major(rows[:1], 128)
    ctx_cm = chunk_major(rows[1:], 3 * 128)
    return jnp.concatenate([vi_cm, ctx_cm], axis=2).reshape(NW, RP_WORDS)

  uu_rows = pack_rows(v_i_uu, v_j_uu, negsamples_uu)
  uu_scalar = uu_rows[0][None] * SIZE_UU + uu_rows[1:]

  idx_all = jnp.concatenate([
      chunk_major(uu_scalar, UU_PAD).reshape(NW, UU_WORDS),
      pack_rp(v_i_up, v_j_up, negsamples_up),
      pack_rp(v_i_pp, v_j_pp, negsamples_pp),
  ], axis=1)

  g_uu = _uu_gram(emb_uu_nodes, emb_uu_ctx)
  dots = _sc_dots((
      g_uu.reshape(-1), emb_up_nodes, emb_up_ctx,
      emb_pp_nodes, emb_pp_ctx, idx_all,
  ))
  fused = _fusion(emb_uu_nodes, emb_up_nodes, emb_pp_nodes,
                  att_fc_W, att_fc_b.reshape(1, D), att_cv_w)
  loss = _loss_epilogue(dots)[0, 0]
  return (loss, fused)
